# baseline (speedup 1.0000x reference)
import functools

import jax
import jax.numpy as jnp
from jax import lax
from jax.experimental import pallas as pl
from jax.experimental.pallas import tpu as pltpu
from jax.experimental.pallas import tpu_sc as plsc

_MAX_TIME_ELAPSED = 300000.0
_MAX_TIME_LAG = 86400.0

_NC = 2
_NS = 16
_NW = _NC * _NS
_LANES = 16
_CH = 256
_NBUF = 2

_GATHER_DNUMS = lax.GatherDimensionNumbers(
    offset_dims=(), collapsed_slice_dims=(0,), start_index_map=(0,))


def _lane_splat(v, idx):
    return lax.gather(v, idx[:, None], _GATHER_DNUMS, (1,),
                      mode=lax.GatherScatterMode.PROMISE_IN_BOUNDS)


_lane_splat_i = _lane_splat


def _sc_body(n_chunks, resp_hbm, el_hbm, la_hbm, wtab_hbm, out_hbm,
             wtab_v, rbuf, ebuf, lbuf, obuf, sem_in, sem_out):
    cid = lax.axis_index("c")
    sid = lax.axis_index("s")
    wid = sid * _NC + cid
    rows_per_w = n_chunks * _CH
    base_row = wid * rows_per_w

    pltpu.sync_copy(wtab_hbm, wtab_v)
    t0b = [wtab_v[0, pl.ds(k * _LANES, _LANES)] for k in range(8)]
    td = [wtab_v[1, pl.ds(k * _LANES, _LANES)] for k in range(8)]
    w0 = [wtab_v[2, pl.ds(k * _LANES, _LANES)] for k in range(8)]
    w1 = [wtab_v[3, pl.ds(k * _LANES, _LANES)] for k in range(8)]

    def in_copies(g, s):
        off = base_row + g * _CH
        sl = pl.ds(off, _CH)
        return (
            pltpu.make_async_copy(resp_hbm.at[sl], rbuf.at[s], sem_in.at[s]),
            pltpu.make_async_copy(el_hbm.at[sl], ebuf.at[s], sem_in.at[s]),
            pltpu.make_async_copy(la_hbm.at[sl], lbuf.at[s], sem_in.at[s]),
        )

    def out_copy(g, s):
        off = base_row + g * _CH
        return pltpu.make_async_copy(
            obuf.at[s], out_hbm.at[pl.ds(off, _CH)], sem_out.at[s])

    def start_in(g, s):
        for cp in in_copies(g, s):
            cp.start()

    def wait_in(g, s):
        for cp in in_copies(g, s):
            cp.wait()

    def compute(s):
        def rowgroup(t, _):
            sl = pl.ds(t * _LANES, _LANES)
            rv = rbuf[s, sl].astype(jnp.float32)
            ev = jnp.clip(ebuf[s, sl] * (1.0 / _MAX_TIME_ELAPSED), 0.0, 1.0)
            lv = jnp.clip(lbuf[s, sl] * (1.0 / _MAX_TIME_LAG), 0.0, 1.0)
            def row(j, _):
                idx = jnp.full((_LANES,), j, jnp.int32)
                rf = _lane_splat(rv, idx)
                e = _lane_splat(ev, idx)
                l = _lane_splat(lv, idx)
                i = t * _LANES + j
                for k in range(8):
                    o = (t0b[k] + rf * td[k]) + (e * w0[k] + l * w1[k])
                    obuf[s, i, pl.ds(k * _LANES, _LANES)] = o
                return 0
            lax.fori_loop(0, _LANES, row, 0, unroll=1)
            return 0
        lax.fori_loop(0, _CH // _LANES, rowgroup, 0)

    for s in range(_NBUF):
        start_in(s, s)

    def outer(g2, _):
        for s in range(_NBUF):
            g = g2 * _NBUF + s
            wait_in(g, s)

            @pl.when(g >= _NBUF)
            def _():
                out_copy(g - _NBUF, s).wait()

            compute(s)
            out_copy(g, s).start()

            @pl.when(g + _NBUF < n_chunks)
            def _():
                start_in(g + _NBUF, s)
        return 0

    lax.fori_loop(0, n_chunks // _NBUF, outer, 0)

    for s in range(_NBUF):
        out_copy(n_chunks - _NBUF + s, s).wait()


def kernel(responses, elapsed_time, lag_time, response_table, time_W, time_b):
    B, L = responses.shape
    D = response_table.shape[1]
    N = B * L
    assert N % (_NW * _CH) == 0
    n_chunks = N // (_NW * _CH)

    wtab = jnp.stack([
        response_table[0] + time_b,
        response_table[1] - response_table[0],
        time_W[0],
        time_W[1],
    ])

    mesh = plsc.VectorSubcoreMesh(core_axis_name="c", subcore_axis_name="s")
    run = pl.kernel(
        functools.partial(_sc_body, n_chunks),
        out_type=jax.ShapeDtypeStruct((N, D), jnp.float32),
        mesh=mesh,
        scratch_types=[
            pltpu.VMEM((4, D), jnp.float32),
            pltpu.VMEM((_NBUF, _CH), jnp.int32),
            pltpu.VMEM((_NBUF, _CH), jnp.float32),
            pltpu.VMEM((_NBUF, _CH), jnp.float32),
            pltpu.VMEM((_NBUF, _CH, D), jnp.float32),
            pltpu.SemaphoreType.DMA((_NBUF,)),
            pltpu.SemaphoreType.DMA((_NBUF,)),
        ],
    )
    out = run(responses.reshape(N), elapsed_time.reshape(N),
              lag_time.reshape(N), wtab)
    return out.reshape(B, L, D)


# speedup vs baseline: 1.0415x; 1.0415x over previous
import functools

import jax
import jax.numpy as jnp
from jax import lax
from jax.experimental import pallas as pl
from jax.experimental.pallas import tpu as pltpu
from jax.experimental.pallas import tpu_sc as plsc

_MAX_TIME_ELAPSED = 300000.0
_MAX_TIME_LAG = 86400.0

_NC = 2
_NS = 16
_NW = _NC * _NS
_LANES = 16
_CH = 256
_NBUF = 2

_GATHER_DNUMS = lax.GatherDimensionNumbers(
    offset_dims=(), collapsed_slice_dims=(0,), start_index_map=(0,))


def _lane_splat(v, idx):
    return lax.gather(v, idx[:, None], _GATHER_DNUMS, (1,),
                      mode=lax.GatherScatterMode.PROMISE_IN_BOUNDS)


def _sc_body(n_chunks, resp_hbm, el_hbm, la_hbm, wtab_hbm, out_hbm,
             wtab_v, rbuf, ebuf, lbuf, obuf, sem_in, sem_out):
    cid = lax.axis_index("c")
    sid = lax.axis_index("s")
    wid = sid * _NC + cid
    rows_per_w = n_chunks * _CH
    base_row = wid * rows_per_w

    pltpu.sync_copy(wtab_hbm, wtab_v)
    t0b = [wtab_v[0, pl.ds(k * _LANES, _LANES)] for k in range(8)]
    td = [wtab_v[1, pl.ds(k * _LANES, _LANES)] for k in range(8)]
    w0 = [wtab_v[2, pl.ds(k * _LANES, _LANES)] for k in range(8)]
    w1 = [wtab_v[3, pl.ds(k * _LANES, _LANES)] for k in range(8)]

    def in_copies(g, s):
        off = base_row + g * _CH
        sl = pl.ds(off, _CH)
        return (
            pltpu.make_async_copy(resp_hbm.at[sl], rbuf.at[s], sem_in.at[s]),
            pltpu.make_async_copy(el_hbm.at[sl], ebuf.at[s], sem_in.at[s]),
            pltpu.make_async_copy(la_hbm.at[sl], lbuf.at[s], sem_in.at[s]),
        )

    def out_copy(g, s):
        off = base_row + g * _CH
        return pltpu.make_async_copy(
            obuf.at[s], out_hbm.at[pl.ds(off, _CH)], sem_out.at[s])

    def start_in(g, s):
        for cp in in_copies(g, s):
            cp.start()

    def wait_in(g, s):
        for cp in in_copies(g, s):
            cp.wait()

    def compute(s):
        def rowgroup(t, _):
            sl = pl.ds(t * _LANES, _LANES)
            rv = rbuf[s, sl].astype(jnp.float32)
            ev = jnp.clip(ebuf[s, sl] * (1.0 / _MAX_TIME_ELAPSED), 0.0, 1.0)
            lv = jnp.clip(lbuf[s, sl] * (1.0 / _MAX_TIME_LAG), 0.0, 1.0)
            def row(j, _):
                idx = jnp.full((_LANES,), j, jnp.int32)
                rf = _lane_splat(rv, idx)
                e = _lane_splat(ev, idx)
                l = _lane_splat(lv, idx)
                i = t * _LANES + j
                for k in range(8):
                    o = t0b[k] + rf * td[k] + e * w0[k] + l * w1[k]
                    obuf[s, i, pl.ds(k * _LANES, _LANES)] = o
                return 0
            lax.fori_loop(0, _LANES, row, 0, unroll=1)
            return 0
        lax.fori_loop(0, _CH // _LANES, rowgroup, 0)

    for s in range(_NBUF):
        start_in(s, s)

    def outer(g2, _):
        for s in range(_NBUF):
            g = g2 * _NBUF + s
            wait_in(g, s)

            @pl.when(g >= _NBUF)
            def _():
                out_copy(g - _NBUF, s).wait()

            compute(s)
            out_copy(g, s).start()

            @pl.when(g + _NBUF < n_chunks)
            def _():
                start_in(g + _NBUF, s)
        return 0

    lax.fori_loop(0, n_chunks // _NBUF, outer, 0)

    for s in range(_NBUF):
        out_copy(n_chunks - _NBUF + s, s).wait()


def kernel(responses, elapsed_time, lag_time, response_table, time_W, time_b):
    B, L = responses.shape
    D = response_table.shape[1]
    N = B * L
    assert N % (_NW * _CH) == 0
    n_chunks = N // (_NW * _CH)

    wtab = jnp.stack([
        response_table[0] + time_b,
        response_table[1] - response_table[0],
        time_W[0],
        time_W[1],
    ])

    mesh = plsc.VectorSubcoreMesh(core_axis_name="c", subcore_axis_name="s")
    run = pl.kernel(
        functools.partial(_sc_body, n_chunks),
        out_type=jax.ShapeDtypeStruct((N, D), jnp.float32),
        mesh=mesh,
        scratch_types=[
            pltpu.VMEM((4, D), jnp.float32),
            pltpu.VMEM((_NBUF, _CH), jnp.int32),
            pltpu.VMEM((_NBUF, _CH), jnp.float32),
            pltpu.VMEM((_NBUF, _CH), jnp.float32),
            pltpu.VMEM((_NBUF, _CH, D), jnp.float32),
            pltpu.SemaphoreType.DMA((_NBUF,)),
            pltpu.SemaphoreType.DMA((_NBUF,)),
        ],
    )
    out = run(responses.reshape(N), elapsed_time.reshape(N),
              lag_time.reshape(N), wtab)
    return out.reshape(B, L, D)


# speedup vs baseline: 1.0476x; 1.0059x over previous
import functools

import jax
import jax.numpy as jnp
from jax import lax
from jax.experimental import pallas as pl
from jax.experimental.pallas import tpu as pltpu
from jax.experimental.pallas import tpu_sc as plsc

_MAX_TIME_ELAPSED = 300000.0
_MAX_TIME_LAG = 86400.0

_NC = 2
_NS = 16
_NW = _NC * _NS
_LANES = 16
_CH = 256
_NBUF = 2

_GATHER_DNUMS = lax.GatherDimensionNumbers(
    offset_dims=(), collapsed_slice_dims=(0,), start_index_map=(0,))


def _lane_splat(v, idx):
    return lax.gather(v, idx[:, None], _GATHER_DNUMS, (1,),
                      mode=lax.GatherScatterMode.PROMISE_IN_BOUNDS)


def _sc_body(n_chunks, resp_hbm, el_hbm, la_hbm, wtab_hbm, out_hbm,
             wtab_v, rbuf, ebuf, lbuf, obuf, sem_in, sem_out):
    cid = lax.axis_index("c")
    sid = lax.axis_index("s")
    wid = sid * _NC + cid
    rows_per_w = n_chunks * _CH
    base_row = wid * rows_per_w

    pltpu.sync_copy(wtab_hbm, wtab_v)
    t0b = [wtab_v[0, pl.ds(k * _LANES, _LANES)] for k in range(8)]
    td = [wtab_v[1, pl.ds(k * _LANES, _LANES)] for k in range(8)]
    w0 = [wtab_v[2, pl.ds(k * _LANES, _LANES)] for k in range(8)]
    w1 = [wtab_v[3, pl.ds(k * _LANES, _LANES)] for k in range(8)]

    def in_copies(g, s):
        off = base_row + g * _CH
        sl = pl.ds(off, _CH)
        return (
            pltpu.make_async_copy(resp_hbm.at[sl], rbuf.at[s], sem_in.at[s]),
            pltpu.make_async_copy(el_hbm.at[sl], ebuf.at[s], sem_in.at[s]),
            pltpu.make_async_copy(la_hbm.at[sl], lbuf.at[s], sem_in.at[s]),
        )

    def out_copy(g, s):
        off = base_row + g * _CH
        return pltpu.make_async_copy(
            obuf.at[s], out_hbm.at[pl.ds(off, _CH)], sem_out.at[s])

    def start_in(g, s):
        for cp in in_copies(g, s):
            cp.start()

    def wait_in(g, s):
        for cp in in_copies(g, s):
            cp.wait()

    def compute(s):
        def rowgroup(t, _):
            sl = pl.ds(t * _LANES, _LANES)
            rv = rbuf[s, sl].astype(jnp.float32)
            ev = jnp.clip(ebuf[s, sl] * (1.0 / _MAX_TIME_ELAPSED), 0.0, 1.0)
            lv = jnp.clip(lbuf[s, sl] * (1.0 / _MAX_TIME_LAG), 0.0, 1.0)
            def row(j, _):
                idx = jnp.full((_LANES,), j, jnp.int32)
                rf = _lane_splat(rv, idx)
                e = _lane_splat(ev, idx)
                l = _lane_splat(lv, idx)
                i = t * _LANES + j
                for k in range(8):
                    o = t0b[k] + rf * td[k] + e * w0[k] + l * w1[k]
                    obuf[s, i, pl.ds(k * _LANES, _LANES)] = o
                return 0
            lax.fori_loop(0, _LANES, row, 0, unroll=1)
            return 0
        lax.fori_loop(0, _CH // _LANES, rowgroup, 0, unroll=2)

    for s in range(_NBUF):
        start_in(s, s)

    def outer(g2, _):
        for s in range(_NBUF):
            g = g2 * _NBUF + s
            wait_in(g, s)

            @pl.when(g >= _NBUF)
            def _():
                out_copy(g - _NBUF, s).wait()

            compute(s)
            out_copy(g, s).start()

            @pl.when(g + _NBUF < n_chunks)
            def _():
                start_in(g + _NBUF, s)
        return 0

    lax.fori_loop(0, n_chunks // _NBUF, outer, 0)

    for s in range(_NBUF):
        out_copy(n_chunks - _NBUF + s, s).wait()


def kernel(responses, elapsed_time, lag_time, response_table, time_W, time_b):
    B, L = responses.shape
    D = response_table.shape[1]
    N = B * L
    assert N % (_NW * _CH) == 0
    n_chunks = N // (_NW * _CH)

    wtab = jnp.stack([
        response_table[0] + time_b,
        response_table[1] - response_table[0],
        time_W[0],
        time_W[1],
    ])

    mesh = plsc.VectorSubcoreMesh(core_axis_name="c", subcore_axis_name="s")
    run = pl.kernel(
        functools.partial(_sc_body, n_chunks),
        out_type=jax.ShapeDtypeStruct((N, D), jnp.float32),
        mesh=mesh,
        scratch_types=[
            pltpu.VMEM((4, D), jnp.float32),
            pltpu.VMEM((_NBUF, _CH), jnp.int32),
            pltpu.VMEM((_NBUF, _CH), jnp.float32),
            pltpu.VMEM((_NBUF, _CH), jnp.float32),
            pltpu.VMEM((_NBUF, _CH, D), jnp.float32),
            pltpu.SemaphoreType.DMA((_NBUF,)),
            pltpu.SemaphoreType.DMA((_NBUF,)),
        ],
    )
    out = run(responses.reshape(N), elapsed_time.reshape(N),
              lag_time.reshape(N), wtab)
    return out.reshape(B, L, D)
